# baseline (speedup 1.0000x reference)
import functools

import jax
import jax.numpy as jnp
from jax import lax
from jax.experimental import pallas as pl
from jax.experimental.pallas import tpu as pltpu
from jax.experimental.pallas import tpu_sc as plsc

_VOCAB = 8192
_CDIM = 256
_COMMIT = 0.25

_MBLK = 512
_BIG = 2**30
_WINDOWS = ((0, 2736), (2736, 2736), (5472, 2720))


def _round_bf16(x):
    u = lax.bitcast_convert_type(x, jnp.uint32)
    u = (u + jnp.uint32(0x7FFF) + ((u >> 16) & jnp.uint32(1))) \
        & jnp.uint32(0xFFFF0000)
    return lax.bitcast_convert_type(u, jnp.float32)


def _argmin_body(zsq_ref, z_ref, csq_ref, cb_ref, idx_ref, loss_ref):
    m = pl.program_id(0)
    zs = (z_ref[...] * 2.0).astype(jnp.bfloat16)
    zsq = zsq_ref[...]

    gv = None
    gvf = None
    gi = None
    for (lo, sz) in _WINDOWS:
        cb = cb_ref[pl.ds(lo, sz), :]
        m2 = lax.dot_general(zs, cb, (((1,), (1,)), ((), ())),
                             preferred_element_type=jnp.float32)
        d = (zsq + csq_ref[:, pl.ds(lo, sz)]) - m2
        lmin = jnp.min(d, axis=1, keepdims=True)
        iota = lax.broadcasted_iota(jnp.int32, (1, sz), 1).astype(jnp.float32)
        lidx = jnp.min(jnp.where(d == lmin, iota, jnp.inf), axis=1,
                       keepdims=True)
        lidx = lidx.astype(jnp.int32) + lo
        if gv is None:
            gv, gvf, gi = _round_bf16(lmin), lmin, lidx
        else:
            take = lmin < gv
            gi = jnp.where(take, lidx, gi)
            gvf = jnp.where(take, lmin, gvf)
            gv = _round_bf16(jnp.where(take, lmin, gv))

    idx_ref[...] = gi
    part = jnp.sum(gvf, keepdims=True)
    prev = jnp.where(m == 0, jnp.zeros_like(loss_ref[...]), loss_ref[...])
    loss_ref[...] = prev + part


def _run_argmin(ze_flat, zsq, codebook_bf16, csq):
    mtot = ze_flat.shape[0]
    grid = (mtot // _MBLK,)
    return pl.pallas_call(
        _argmin_body,
        grid=grid,
        in_specs=[
            pl.BlockSpec((_MBLK, 1), lambda i: (i, 0)),
            pl.BlockSpec((_MBLK, _CDIM), lambda i: (i, 0)),
            pl.BlockSpec((1, _VOCAB), lambda i: (0, 0)),
            pl.BlockSpec((_VOCAB, _CDIM), lambda i: (0, 0)),
        ],
        out_specs=[
            pl.BlockSpec((_MBLK, 1), lambda i: (i, 0)),
            pl.BlockSpec((1, 1), lambda i: (0, 0)),
        ],
        out_shape=[
            jax.ShapeDtypeStruct((mtot, 1), jnp.int32),
            jax.ShapeDtypeStruct((1, 1), jnp.float32),
        ],
    )(zsq, ze_flat, csq, codebook_bf16)


@functools.cache
def _make_sc_gather(nrows):
    nw = 32
    chunk = 128
    bper = nrows // nw
    nch = bper // chunk

    def body(table_hbm, idx_hbm, out_hbm, idx_v, rows0, rows1, sem0, sem1):
        wid = lax.axis_index("s") * 2 + lax.axis_index("c")
        base = wid * bper
        for j in range(nch):
            pltpu.sync_copy(idx_hbm.at[pl.ds(base + j * chunk, chunk)],
                            idx_v.at[j])
        bufs = (rows0, rows1)
        sems = (sem0, sem1)
        cps = {}
        cps[0] = pltpu.async_copy(table_hbm.at[idx_v.at[0]], bufs[0], sems[0])
        for j in range(nch):
            if j + 1 < nch:
                cps[j + 1] = pltpu.async_copy(
                    table_hbm.at[idx_v.at[j + 1]], bufs[(j + 1) % 2],
                    sems[(j + 1) % 2])
            cps[j].wait()
            pltpu.sync_copy(bufs[j % 2],
                            out_hbm.at[pl.ds(base + j * chunk, chunk)])

    return functools.partial(
        pl.kernel,
        mesh=plsc.VectorSubcoreMesh(core_axis_name="c", subcore_axis_name="s"),
        out_type=jax.ShapeDtypeStruct((nrows, _CDIM), jnp.float32),
        scratch_types=[
            pltpu.VMEM((nch, chunk), jnp.int32),
            pltpu.VMEM((chunk, _CDIM), jnp.float32),
            pltpu.VMEM((chunk, _CDIM), jnp.float32),
            pltpu.SemaphoreType.DMA,
            pltpu.SemaphoreType.DMA,
        ],
    )(body)


def kernel(ze, codebook):
    zep = jnp.transpose(ze, (0, 2, 3, 1))
    B, H, W, C = zep.shape
    ze_flat = zep.reshape(B * H * W, C)
    zsq = jnp.sum(ze_flat ** 2, axis=1, keepdims=True)
    csq = jnp.sum(codebook ** 2, axis=1).reshape(1, _VOCAB)

    cb16 = codebook.astype(jnp.bfloat16)
    n = B * H * W
    half = n // 2
    gather = _make_sc_gather(half)

    idx_a, loss_a = _run_argmin(ze_flat[:half], zsq[:half], cb16, csq)
    zq_a = gather(codebook, idx_a.reshape(half))
    idx_b, loss_b = _run_argmin(ze_flat[half:], zsq[half:], cb16, csq)
    zq_b = gather(codebook, idx_b.reshape(half))

    zq = jnp.concatenate([zq_a, zq_b], axis=0).reshape(B, H, W, C)
    loss_sum = loss_a[0, 0] + loss_b[0, 0]
    vq_loss = loss_sum * ((1.0 + _COMMIT) / (B * H * W * C))
    zq_out = jnp.transpose(zq, (0, 3, 1, 2))
    idx2d = jnp.concatenate([idx_a, idx_b], axis=0)
    return (vq_loss, zq_out, idx2d.reshape(B, H * W))


# speedup vs baseline: 1.5057x; 1.5057x over previous
import functools

import jax
import jax.numpy as jnp
from jax import lax
from jax.experimental import pallas as pl
from jax.experimental.pallas import tpu as pltpu
from jax.experimental.pallas import tpu_sc as plsc

_VOCAB = 8192
_CDIM = 256
_COMMIT = 0.25

_MBLK = 512
_BIG = 2**30
_WINDOWS = ((0, 2736), (2736, 2736), (5472, 2720))


def _round_bf16(x):
    u = lax.bitcast_convert_type(x, jnp.uint32)
    u = (u + jnp.uint32(0x7FFF) + ((u >> 16) & jnp.uint32(1))) \
        & jnp.uint32(0xFFFF0000)
    return lax.bitcast_convert_type(u, jnp.float32)


def _argmin_body(zsq_ref, zs_ref, csq_ref, cb_ref, idx_ref, loss_ref):
    m = pl.program_id(0)
    zsq = zsq_ref[...]
    m2 = lax.dot_general(zs_ref[...], cb_ref[...], (((1,), (1,)), ((), ())),
                         preferred_element_type=jnp.float32)
    d = (zsq + csq_ref[...]) - m2

    gv = None
    gvf = None
    gi = None
    for (lo, sz) in _WINDOWS:
        dw = d[:, lo:lo + sz]
        lmin = jnp.min(dw, axis=1, keepdims=True)
        iota = lax.broadcasted_iota(jnp.int32, (1, sz), 1).astype(jnp.float32)
        lidx = jnp.min(jnp.where(dw == lmin, iota, jnp.inf), axis=1,
                       keepdims=True)
        lidx = lidx.astype(jnp.int32) + lo
        if gv is None:
            gv, gvf, gi = _round_bf16(lmin), lmin, lidx
        else:
            take = lmin < gv
            gi = jnp.where(take, lidx, gi)
            gvf = jnp.where(take, lmin, gvf)
            gv = _round_bf16(jnp.where(take, lmin, gv))

    idx_ref[...] = gi
    part = jnp.sum(gvf, keepdims=True)
    prev = jnp.where(m == 0, jnp.zeros_like(loss_ref[...]), loss_ref[...])
    loss_ref[...] = prev + part


def _run_argmin(zs_bf16, zsq, codebook_bf16, csq):
    mtot = zs_bf16.shape[0]
    grid = (mtot // _MBLK,)
    return pl.pallas_call(
        _argmin_body,
        grid=grid,
        in_specs=[
            pl.BlockSpec((_MBLK, 1), lambda i: (i, 0)),
            pl.BlockSpec((_MBLK, _CDIM), lambda i: (i, 0)),
            pl.BlockSpec((1, _VOCAB), lambda i: (0, 0)),
            pl.BlockSpec((_VOCAB, _CDIM), lambda i: (0, 0)),
        ],
        out_specs=[
            pl.BlockSpec((_MBLK, 1), lambda i: (i, 0)),
            pl.BlockSpec((1, 1), lambda i: (0, 0)),
        ],
        out_shape=[
            jax.ShapeDtypeStruct((mtot, 1), jnp.int32),
            jax.ShapeDtypeStruct((1, 1), jnp.float32),
        ],
    )(zsq, zs_bf16, csq, codebook_bf16)


@functools.cache
def _make_sc_gather(nrows):
    nw = 32
    chunk = 128
    bper = nrows // nw
    nch = bper // chunk

    def body(table_hbm, idx_hbm, out_hbm, idx_v, rows0, rows1, sem0, sem1):
        wid = lax.axis_index("s") * 2 + lax.axis_index("c")
        base = wid * bper
        for j in range(nch):
            pltpu.sync_copy(idx_hbm.at[pl.ds(base + j * chunk, chunk)],
                            idx_v.at[j])
        bufs = (rows0, rows1)
        sems = (sem0, sem1)
        cps = {}
        cps[0] = pltpu.async_copy(table_hbm.at[idx_v.at[0]], bufs[0], sems[0])
        for j in range(nch):
            if j + 1 < nch:
                cps[j + 1] = pltpu.async_copy(
                    table_hbm.at[idx_v.at[j + 1]], bufs[(j + 1) % 2],
                    sems[(j + 1) % 2])
            cps[j].wait()
            pltpu.sync_copy(bufs[j % 2],
                            out_hbm.at[pl.ds(base + j * chunk, chunk)])

    return functools.partial(
        pl.kernel,
        mesh=plsc.VectorSubcoreMesh(core_axis_name="c", subcore_axis_name="s"),
        out_type=jax.ShapeDtypeStruct((nrows, _CDIM), jnp.float32),
        scratch_types=[
            pltpu.VMEM((nch, chunk), jnp.int32),
            pltpu.VMEM((chunk, _CDIM), jnp.float32),
            pltpu.VMEM((chunk, _CDIM), jnp.float32),
            pltpu.SemaphoreType.DMA,
            pltpu.SemaphoreType.DMA,
        ],
    )(body)


def kernel(ze, codebook):
    zep = jnp.transpose(ze, (0, 2, 3, 1))
    B, H, W, C = zep.shape
    ze_flat = zep.reshape(B * H * W, C)
    zsq = jnp.sum(ze_flat ** 2, axis=1, keepdims=True)
    csq = jnp.sum(codebook ** 2, axis=1).reshape(1, _VOCAB)

    cb16 = codebook.astype(jnp.bfloat16)
    zs16 = (2.0 * ze_flat).astype(jnp.bfloat16)
    n = B * H * W

    idx2d, loss_sum = _run_argmin(zs16, zsq, cb16, csq)
    zq_flat = _make_sc_gather(n)(codebook, idx2d.reshape(n))

    zq = zq_flat.reshape(B, H, W, C)
    vq_loss = loss_sum[0, 0] * ((1.0 + _COMMIT) / (B * H * W * C))
    zq_out = jnp.transpose(zq, (0, 3, 1, 2))
    return (vq_loss, zq_out, idx2d.reshape(B, H * W))


# speedup vs baseline: 1.5075x; 1.0012x over previous
import functools

import jax
import jax.numpy as jnp
from jax import lax
from jax.experimental import pallas as pl
from jax.experimental.pallas import tpu as pltpu
from jax.experimental.pallas import tpu_sc as plsc

_VOCAB = 8192
_CDIM = 256
_COMMIT = 0.25

_MBLK = 512
_BIG = 2**30
_WINDOWS = ((0, 2736), (2736, 2736), (5472, 2720))


def _round_bf16(x):
    u = lax.bitcast_convert_type(x, jnp.uint32)
    u = (u + jnp.uint32(0x7FFF) + ((u >> 16) & jnp.uint32(1))) \
        & jnp.uint32(0xFFFF0000)
    return lax.bitcast_convert_type(u, jnp.float32)


def _argmin_body(zsq_ref, zs_ref, csq_ref, cb_ref, idx_ref, loss_ref):
    m = pl.program_id(0)
    zsq = zsq_ref[...]
    m2 = lax.dot_general(zs_ref[...], cb_ref[...], (((1,), (1,)), ((), ())),
                         preferred_element_type=jnp.float32)
    d = (zsq + csq_ref[...]) - m2

    gv = None
    gvf = None
    gi = None
    for (lo, sz) in _WINDOWS:
        dw = d[:, lo:lo + sz]
        lmin = jnp.min(dw, axis=1, keepdims=True)
        iota = lax.broadcasted_iota(jnp.int32, (1, sz), 1).astype(jnp.float32)
        lidx = jnp.min(jnp.where(dw == lmin, iota, jnp.inf), axis=1,
                       keepdims=True)
        lidx = lidx.astype(jnp.int32) + lo
        if gv is None:
            gv, gvf, gi = _round_bf16(lmin), lmin, lidx
        else:
            take = lmin < gv
            gi = jnp.where(take, lidx, gi)
            gvf = jnp.where(take, lmin, gvf)
            gv = _round_bf16(jnp.where(take, lmin, gv))

    idx_ref[...] = gi
    part = jnp.sum(gvf, keepdims=True)
    prev = jnp.where(m == 0, jnp.zeros_like(loss_ref[...]), loss_ref[...])
    loss_ref[...] = prev + part


def _run_argmin(zs_bf16, zsq, codebook_bf16, csq):
    mtot = zs_bf16.shape[0]
    grid = (mtot // _MBLK,)
    return pl.pallas_call(
        _argmin_body,
        grid=grid,
        in_specs=[
            pl.BlockSpec((_MBLK, 1), lambda i: (i, 0)),
            pl.BlockSpec((_MBLK, _CDIM), lambda i: (i, 0)),
            pl.BlockSpec((1, _VOCAB), lambda i: (0, 0)),
            pl.BlockSpec((_VOCAB, _CDIM), lambda i: (0, 0)),
        ],
        out_specs=[
            pl.BlockSpec((_MBLK, 1), lambda i: (i, 0)),
            pl.BlockSpec((1, 1), lambda i: (0, 0)),
        ],
        out_shape=[
            jax.ShapeDtypeStruct((mtot, 1), jnp.int32),
            jax.ShapeDtypeStruct((1, 1), jnp.float32),
        ],
    )(zsq, zs_bf16, csq, codebook_bf16)


@functools.cache
def _make_sc_gather(nrows):
    nw = 32
    chunk = 128
    bper = nrows // nw
    nch = bper // chunk

    def body(table_hbm, idx_hbm, out_hbm, idx_v, rows0, rows1, rows2,
             gs0, gs1, gs2, ws0, ws1, ws2):
        wid = lax.axis_index("s") * 2 + lax.axis_index("c")
        base = wid * bper
        for j in range(nch):
            pltpu.sync_copy(idx_hbm.at[pl.ds(base + j * chunk, chunk)],
                            idx_v.at[j])
        bufs = (rows0, rows1, rows2)
        gsems = (gs0, gs1, gs2)
        wsems = (ws0, ws1, ws2)
        nbuf = 3
        gcp, wcp = {}, {}
        for j in range(min(nbuf, nch)):
            gcp[j] = pltpu.async_copy(table_hbm.at[idx_v.at[j]],
                                      bufs[j % nbuf], gsems[j % nbuf])
        for j in range(nch):
            gcp[j].wait()
            wcp[j] = pltpu.async_copy(
                bufs[j % nbuf], out_hbm.at[pl.ds(base + j * chunk, chunk)],
                wsems[j % nbuf])
            nxt = j + nbuf
            if nxt < nch:
                wcp[j].wait()
                gcp[nxt] = pltpu.async_copy(table_hbm.at[idx_v.at[nxt]],
                                            bufs[nxt % nbuf],
                                            gsems[nxt % nbuf])
        for j in range(max(0, nch - nbuf), nch):
            wcp[j].wait()

    return functools.partial(
        pl.kernel,
        mesh=plsc.VectorSubcoreMesh(core_axis_name="c", subcore_axis_name="s"),
        out_type=jax.ShapeDtypeStruct((nrows, _CDIM), jnp.float32),
        scratch_types=[
            pltpu.VMEM((nch, chunk), jnp.int32),
            pltpu.VMEM((chunk, _CDIM), jnp.float32),
            pltpu.VMEM((chunk, _CDIM), jnp.float32),
            pltpu.VMEM((chunk, _CDIM), jnp.float32),
            pltpu.SemaphoreType.DMA,
            pltpu.SemaphoreType.DMA,
            pltpu.SemaphoreType.DMA,
            pltpu.SemaphoreType.DMA,
            pltpu.SemaphoreType.DMA,
            pltpu.SemaphoreType.DMA,
        ],
    )(body)


def kernel(ze, codebook):
    zep = jnp.transpose(ze, (0, 2, 3, 1))
    B, H, W, C = zep.shape
    ze_flat = zep.reshape(B * H * W, C)
    zsq = jnp.sum(ze_flat ** 2, axis=1, keepdims=True)
    csq = jnp.sum(codebook ** 2, axis=1).reshape(1, _VOCAB)

    cb16 = codebook.astype(jnp.bfloat16)
    zs16 = (2.0 * ze_flat).astype(jnp.bfloat16)
    n = B * H * W

    idx2d, loss_sum = _run_argmin(zs16, zsq, cb16, csq)
    zq_flat = _make_sc_gather(n)(codebook, idx2d.reshape(n))

    zq = zq_flat.reshape(B, H, W, C)
    vq_loss = loss_sum[0, 0] * ((1.0 + _COMMIT) / (B * H * W * C))
    zq_out = jnp.transpose(zq, (0, 3, 1, 2))
    return (vq_loss, zq_out, idx2d.reshape(B, H * W))
